# baseline (speedup 1.0000x reference)
import functools

import jax
import jax.numpy as jnp
from jax import lax
from jax.experimental import pallas as pl
from jax.experimental.pallas import tpu as pltpu
from jax.experimental.pallas import tpu_sc as plsc

B, S, D = 4096, 50, 128
NC, NS = 2, 16
NW = NC * NS
K = 4
BSLAB = B // K
NSTEP = BSLAB // NW
NBUF = 8
NGRP = NSTEP // NBUF


def _emb_body(x_hbm, w_hbm, out_hbm, idx_v, rows_v, gsems, osems):
    wid = lax.axis_index("s") * NC + lax.axis_index("c")
    b0 = wid * NSTEP

    pltpu.sync_copy(x_hbm.at[wid], idx_v)

    def gstart(b, step):
        pltpu.async_copy(w_hbm.at[idx_v.at[step]], rows_v.at[b], gsems.at[b])

    def gwait(b):
        pltpu.make_async_copy(w_hbm.at[idx_v.at[0]], rows_v.at[b],
                              gsems.at[b]).wait()

    def wstart(b, step):
        pltpu.async_copy(rows_v.at[b], out_hbm.at[b0 + step], osems.at[b])

    def wwait(b):
        pltpu.make_async_copy(rows_v.at[b], out_hbm.at[b0],
                              osems.at[b]).wait()

    for b in range(NBUF):
        gstart(b, b)

    def group(g, _):
        for b in range(NBUF):
            gwait(b)
            wstart(b, g * NBUF + b)
        for b in range(NBUF):
            wwait(b)
            nxt = (g + 1) * NBUF + b

            @pl.when(g < NGRP - 1)
            def _():
                gstart(b, nxt)
        return _

    lax.fori_loop(0, NGRP, group, None)


@jax.jit
def _emb(xw, weight):
    kern = pl.kernel(
        _emb_body,
        out_type=jax.ShapeDtypeStruct((BSLAB, S, D), jnp.float32),
        mesh=plsc.VectorSubcoreMesh(core_axis_name="c", subcore_axis_name="s"),
        compiler_params=pltpu.CompilerParams(use_tc_tiling_on_sc=True),
        scratch_types=[
            pltpu.VMEM((NSTEP, S), jnp.int32),
            pltpu.VMEM((NBUF, S, D), jnp.float32),
            pltpu.SemaphoreType.DMA((NBUF,)),
            pltpu.SemaphoreType.DMA((NBUF,)),
        ],
    )
    outs = [kern(xw[k], weight) for k in range(K)]
    return jnp.concatenate(outs, axis=0)


def kernel(x, weight):
    xw = x.astype(jnp.int32).reshape(K, NW, NSTEP, S)
    return _emb(xw, weight)


# speedup vs baseline: 3.1630x; 3.1630x over previous
import functools

import jax
import jax.numpy as jnp
from jax import lax
from jax.experimental import pallas as pl
from jax.experimental.pallas import tpu as pltpu
from jax.experimental.pallas import tpu_sc as plsc

B, S, D = 4096, 50, 128
NC, NS = 2, 16
NW = NC * NS
BPW = B // NW
NBUF = 5
NGRP = S // NBUF


def _emb_body(x_hbm, w_hbm, out_hbm, idx_v, rows_v, gsems, osems):
    wid = lax.axis_index("s") * NC + lax.axis_index("c")
    b0 = wid * BPW

    pltpu.sync_copy(x_hbm.at[wid], idx_v)

    def gstart(b, step):
        pltpu.async_copy(w_hbm.at[idx_v.at[step]], rows_v.at[b], gsems.at[b])

    def gwait(b):
        pltpu.make_async_copy(w_hbm.at[idx_v.at[0]], rows_v.at[b],
                              gsems.at[b]).wait()

    def wstart(b, step):
        pltpu.async_copy(rows_v.at[b], out_hbm.at[step, pl.ds(b0, BPW)],
                         osems.at[b])

    def wwait(b):
        pltpu.make_async_copy(rows_v.at[b], out_hbm.at[0, pl.ds(b0, BPW)],
                              osems.at[b]).wait()

    for b in range(NBUF):
        gstart(b, b)

    def group(g, _):
        for b in range(NBUF):
            gwait(b)
            wstart(b, g * NBUF + b)
        for b in range(NBUF):
            wwait(b)
            nxt = (g + 1) * NBUF + b

            @pl.when(g < NGRP - 1)
            def _():
                gstart(b, nxt)
        return _

    lax.fori_loop(0, NGRP, group, None)


@jax.jit
def _emb(xw, weight):
    kern = pl.kernel(
        _emb_body,
        out_type=jax.ShapeDtypeStruct((S, B, D), jnp.float32),
        mesh=plsc.VectorSubcoreMesh(core_axis_name="c", subcore_axis_name="s"),
        compiler_params=pltpu.CompilerParams(use_tc_tiling_on_sc=True),
        scratch_types=[
            pltpu.VMEM((S, BPW), jnp.int32),
            pltpu.VMEM((NBUF, BPW, D), jnp.float32),
            pltpu.SemaphoreType.DMA((NBUF,)),
            pltpu.SemaphoreType.DMA((NBUF,)),
        ],
    )
    return kern(xw, weight).transpose(1, 0, 2)


def kernel(x, weight):
    xw = x.astype(jnp.int32).reshape(NW, BPW, S).transpose(0, 2, 1)
    return _emb(xw, weight)


# speedup vs baseline: 3.2166x; 1.0170x over previous
import functools

import jax
import jax.numpy as jnp
from jax import lax
from jax.experimental import pallas as pl
from jax.experimental.pallas import tpu as pltpu
from jax.experimental.pallas import tpu_sc as plsc

B, S, D = 4096, 50, 128
NC, NS = 2, 16
NW = NC * NS
BPW = B // NW
CH = 64
NSTEP = S * BPW // CH
NBUF = 10
NGRP = NSTEP // NBUF


def _emb_body(x_hbm, w_hbm, out_hbm, idx_v, rows_v, gsems, osems):
    wid = lax.axis_index("s") * NC + lax.axis_index("c")
    b0 = wid * BPW

    pltpu.sync_copy(x_hbm.at[wid], idx_v)

    def gstart(b, step):
        pltpu.async_copy(w_hbm.at[idx_v.at[step // 2, pl.ds((step % 2) * CH, CH)]],
                         rows_v.at[b], gsems.at[b])

    def gwait(b):
        pltpu.make_async_copy(w_hbm.at[idx_v.at[0, pl.ds(0, CH)]], rows_v.at[b],
                              gsems.at[b]).wait()

    def wstart(b, step):
        pltpu.async_copy(rows_v.at[b],
                         out_hbm.at[step // 2, pl.ds(b0 + (step % 2) * CH, CH)],
                         osems.at[b])

    def wwait(b):
        pltpu.make_async_copy(rows_v.at[b], out_hbm.at[0, pl.ds(b0, CH)],
                              osems.at[b]).wait()

    for b in range(NBUF):
        gstart(b, b)

    def group(g, _):
        for b in range(NBUF):
            gwait(b)
            wstart(b, g * NBUF + b)
        for b in range(NBUF):
            wwait(b)
            nxt = (g + 1) * NBUF + b

            @pl.when(g < NGRP - 1)
            def _():
                gstart(b, nxt)
        return _

    lax.fori_loop(0, NGRP, group, None)


@jax.jit
def _emb(xw, weight):
    kern = pl.kernel(
        _emb_body,
        out_type=jax.ShapeDtypeStruct((S, B, D), jnp.float32),
        mesh=plsc.VectorSubcoreMesh(core_axis_name="c", subcore_axis_name="s"),
        compiler_params=pltpu.CompilerParams(use_tc_tiling_on_sc=True),
        scratch_types=[
            pltpu.VMEM((S, BPW), jnp.int32),
            pltpu.VMEM((NBUF, CH, D), jnp.float32),
            pltpu.SemaphoreType.DMA((NBUF,)),
            pltpu.SemaphoreType.DMA((NBUF,)),
        ],
    )
    return kern(xw, weight).transpose(1, 0, 2)


def kernel(x, weight):
    xw = x.astype(jnp.int32).reshape(NW, BPW, S).transpose(0, 2, 1)
    return _emb(xw, weight)


# speedup vs baseline: 3.2254x; 1.0027x over previous
import jax
import jax.numpy as jnp
from jax import lax
from jax.experimental import pallas as pl
from jax.experimental.pallas import tpu as pltpu
from jax.experimental.pallas import tpu_sc as plsc

B, S, D = 4096, 50, 128
NC, NS = 2, 16
NW = NC * NS
BPW = B // NW
CH = 64
NSTEP = S * BPW // CH
NBUF = 10
NGRP = NSTEP // NBUF


def _emb_body(x_hbm, w_hbm, out_hbm, idx_v, rows_v, gsems, osems):
    wid = lax.axis_index("s") * NC + lax.axis_index("c")
    b0 = wid * BPW

    pltpu.sync_copy(x_hbm.at[wid], idx_v)

    def gstart(b, step):
        pltpu.async_copy(w_hbm.at[idx_v.at[step // 2, pl.ds((step % 2) * CH, CH)]],
                         rows_v.at[b], gsems.at[b])

    def gwait(b):
        pltpu.make_async_copy(w_hbm.at[idx_v.at[0, pl.ds(0, CH)]], rows_v.at[b],
                              gsems.at[b]).wait()

    def wstart(b, step):
        pltpu.async_copy(rows_v.at[b],
                         out_hbm.at[step // 2, pl.ds(b0 + (step % 2) * CH, CH)],
                         osems.at[b])

    def wwait(b):
        pltpu.make_async_copy(rows_v.at[b], out_hbm.at[0, pl.ds(b0, CH)],
                              osems.at[b]).wait()

    for b in range(NBUF):
        gstart(b, b)

    def group(g, _):
        for b in range(NBUF):
            gwait(b)
            wstart(b, g * NBUF + b)
        for b in range(NBUF):
            wwait(b)
            nxt = (g + 1) * NBUF + b

            @pl.when(g < NGRP - 1)
            def _():
                gstart(b, nxt)
        return _

    lax.fori_loop(0, NGRP, group, None)


@jax.jit
def _emb(xw, weight):
    kern = pl.kernel(
        _emb_body,
        out_type=jax.ShapeDtypeStruct((S, B, D), jnp.float32),
        mesh=plsc.VectorSubcoreMesh(core_axis_name="c", subcore_axis_name="s"),
        compiler_params=pltpu.CompilerParams(use_tc_tiling_on_sc=True),
        scratch_types=[
            pltpu.VMEM((S, BPW), jnp.int32),
            pltpu.VMEM((NBUF, CH, D), jnp.float32),
            pltpu.SemaphoreType.DMA((NBUF,)),
            pltpu.SemaphoreType.DMA((NBUF,)),
        ],
    )
    return kern(xw, weight).transpose(1, 0, 2)


def kernel(x, weight):
    xw = x.astype(jnp.int32).reshape(NW, BPW, S).transpose(0, 2, 1)
    return _emb(xw, weight)
